# initial kernel scaffold (unmeasured)
import jax
import jax.numpy as jnp
from jax import lax
from jax.experimental import pallas as pl
from jax.experimental.pallas import tpu as pltpu

N_DEV = 8


def kernel(x, w_mat):
    m_tot, k_per = x.shape
    _, n_tot = w_mat.shape
    m_per = m_tot // N_DEV
    n_half = n_tot // 2

    def body(x_ref, w_ref, out_ref, comm_ref, send_sems, recv_sems,
             credit_sem, out_dma_sem):
        d = lax.axis_index("i")
        right = jnp.mod(d + 1, N_DEV)
        left = jnp.mod(d - 1, N_DEV)

        barrier = pltpu.get_barrier_semaphore()
        for nbr in (left, right):
            pl.semaphore_signal(barrier, inc=1, device_id=(nbr,),
                                device_id_type=pl.DeviceIdType.MESH)
        pl.semaphore_wait(barrier, 2)

        def chunk(j, half):
            a = x_ref[pl.ds(j * m_per, m_per), :]
            b = w_ref[:, half * n_half:(half + 1) * n_half]
            return jnp.dot(a, b, preferred_element_type=jnp.float32)

        def ring(half, fwd):
            dst = right if fwd else left
            credit_to = left if fwd else right

            def idx(s):
                return jnp.mod(d - 2 - s, N_DEV) if fwd else jnp.mod(d + 2 + s, N_DEV)

            first = jnp.mod(d - 1, N_DEV) if fwd else jnp.mod(d + 1, N_DEV)
            comm_ref[0] = chunk(first, half)

            for s in range(N_DEV - 1):
                ss, rs = s % 2, (s + 1) % 2
                if s >= 2:
                    pl.semaphore_wait(credit_sem, 1)
                rdma = pltpu.make_async_remote_copy(
                    src_ref=comm_ref.at[ss],
                    dst_ref=comm_ref.at[rs],
                    send_sem=send_sems.at[ss],
                    recv_sem=recv_sems.at[rs],
                    device_id=(dst,),
                    device_id_type=pl.DeviceIdType.MESH,
                )
                rdma.start()
                rdma.wait()
                if 1 <= s <= 5:
                    pl.semaphore_signal(credit_sem, inc=1, device_id=(credit_to,),
                                        device_id_type=pl.DeviceIdType.MESH)
                if s < N_DEV - 2:
                    comm_ref[rs] = comm_ref[rs] + chunk(idx(s), half)
                else:
                    comm_ref[ss] = jnp.maximum(comm_ref[rs] + chunk(d, half), 0.0)
                    cp = pltpu.make_async_copy(
                        comm_ref.at[ss],
                        out_ref.at[:, pl.ds(half * n_half, n_half)],
                        out_dma_sem,
                    )
                    cp.start()
                    cp.wait()

        ring(0, True)

        for nbr in (left, right):
            pl.semaphore_signal(barrier, inc=1, device_id=(nbr,),
                                device_id_type=pl.DeviceIdType.MESH)
        pl.semaphore_wait(barrier, 2)

        ring(1, False)

    return pl.pallas_call(
        body,
        out_shape=jax.ShapeDtypeStruct((m_per, n_tot), jnp.float32),
        in_specs=[
            pl.BlockSpec(memory_space=pltpu.VMEM),
            pl.BlockSpec(memory_space=pltpu.VMEM),
        ],
        out_specs=pl.BlockSpec(memory_space=pltpu.ANY),
        scratch_shapes=[
            pltpu.VMEM((2, m_per, n_half), jnp.float32),
            pltpu.SemaphoreType.DMA((2,)),
            pltpu.SemaphoreType.DMA((2,)),
            pltpu.SemaphoreType.REGULAR,
            pltpu.SemaphoreType.DMA,
        ],
        compiler_params=pltpu.CompilerParams(
            collective_id=0,
            vmem_limit_bytes=64 * 1024 * 1024,
        ),
    )(x, w_mat)


# baseline (device time: 1355265 ns/iter reference)
import jax
import jax.numpy as jnp
from jax import lax
from jax.experimental import pallas as pl
from jax.experimental.pallas import tpu as pltpu

N_DEV = 8


def kernel(x, w_mat):
    m_tot, k_per = x.shape
    _, n_tot = w_mat.shape
    m_per = m_tot // N_DEV
    n_half = n_tot // 2

    def body(x_ref, w_ref, out_ref, comm_ref, send_sems, recv_sems,
             credit_sem, out_dma_sem):
        d = lax.axis_index("i")
        right = jnp.mod(d + 1, N_DEV)
        left = jnp.mod(d - 1, N_DEV)

        barrier = pltpu.get_barrier_semaphore()
        for nbr in (left, right):
            pl.semaphore_signal(barrier, inc=1, device_id=(nbr,),
                                device_id_type=pl.DeviceIdType.MESH)
        pl.semaphore_wait(barrier, 2)

        def chunk(j, half):
            a = x_ref[pl.ds(j * m_per, m_per), :]
            b = w_ref[:, half * n_half:(half + 1) * n_half]
            return jnp.dot(a, b, preferred_element_type=jnp.float32)

        def ring(half, fwd):
            dst = right if fwd else left
            credit_to = left if fwd else right

            def idx(s):
                return jnp.mod(d - 2 - s, N_DEV) if fwd else jnp.mod(d + 2 + s, N_DEV)

            first = jnp.mod(d - 1, N_DEV) if fwd else jnp.mod(d + 1, N_DEV)
            comm_ref[0] = chunk(first, half)

            for s in range(N_DEV - 1):
                ss, rs = s % 2, (s + 1) % 2
                if s >= 2:
                    pl.semaphore_wait(credit_sem, 1)
                rdma = pltpu.make_async_remote_copy(
                    src_ref=comm_ref.at[ss],
                    dst_ref=comm_ref.at[rs],
                    send_sem=send_sems.at[ss],
                    recv_sem=recv_sems.at[rs],
                    device_id=(dst,),
                    device_id_type=pl.DeviceIdType.MESH,
                )
                rdma.start()
                rdma.wait()
                if 1 <= s <= 5:
                    pl.semaphore_signal(credit_sem, inc=1, device_id=(credit_to,),
                                        device_id_type=pl.DeviceIdType.MESH)
                if s < N_DEV - 2:
                    comm_ref[rs] = comm_ref[rs] + chunk(idx(s), half)
                else:
                    comm_ref[ss] = jnp.maximum(comm_ref[rs] + chunk(d, half), 0.0)
                    cp = pltpu.make_async_copy(
                        comm_ref.at[ss],
                        out_ref.at[:, pl.ds(half * n_half, n_half)],
                        out_dma_sem,
                    )
                    cp.start()
                    cp.wait()

        ring(0, True)

        for nbr in (left, right):
            pl.semaphore_signal(barrier, inc=1, device_id=(nbr,),
                                device_id_type=pl.DeviceIdType.MESH)
        pl.semaphore_wait(barrier, 2)

        ring(1, False)

    return pl.pallas_call(
        body,
        out_shape=jax.ShapeDtypeStruct((m_per, n_tot), jnp.float32),
        in_specs=[
            pl.BlockSpec(memory_space=pltpu.VMEM),
            pl.BlockSpec(memory_space=pltpu.VMEM),
        ],
        out_specs=pl.BlockSpec(memory_space=pl.ANY),
        scratch_shapes=[
            pltpu.VMEM((2, m_per, n_half), jnp.float32),
            pltpu.SemaphoreType.DMA((2,)),
            pltpu.SemaphoreType.DMA((2,)),
            pltpu.SemaphoreType.REGULAR,
            pltpu.SemaphoreType.DMA,
        ],
        compiler_params=pltpu.CompilerParams(
            collective_id=0,
            vmem_limit_bytes=64 * 1024 * 1024,
        ),
    )(x, w_mat)


# device time: 714607 ns/iter; 1.8965x vs baseline; 1.8965x over previous
import jax
import jax.numpy as jnp
from jax import lax
from jax.experimental import pallas as pl
from jax.experimental.pallas import tpu as pltpu

N_DEV = 8


def kernel(x, w_mat):
    m_tot, k_per = x.shape
    _, n_tot = w_mat.shape
    m_per = m_tot // N_DEV
    n_half = n_tot // 2

    def body(x_ref, w_ref, out_ref, cw_ref, ccw_ref,
             send_cw, recv_cw, send_ccw, recv_ccw,
             credit_cw, credit_ccw, out_sems):
        d = lax.axis_index("i")
        right = jnp.mod(d + 1, N_DEV)
        left = jnp.mod(d - 1, N_DEV)

        barrier = pltpu.get_barrier_semaphore()
        for nbr in (left, right):
            pl.semaphore_signal(barrier, inc=1, device_id=(nbr,),
                                device_id_type=pl.DeviceIdType.MESH)
        pl.semaphore_wait(barrier, 2)

        def chunk(j, half):
            a = x_ref[pl.ds(j * m_per, m_per), :]
            b = w_ref[:, half * n_half:(half + 1) * n_half]
            return jnp.dot(a, b, preferred_element_type=jnp.float32)

        cw_ref[0] = chunk(jnp.mod(d - 1, N_DEV), 0)
        ccw_ref[0] = chunk(jnp.mod(d + 1, N_DEV), 1)

        for s in range(N_DEV - 1):
            ss, rs = s % 2, (s + 1) % 2
            if s >= 2:
                pl.semaphore_wait(credit_cw, 1)
                pl.semaphore_wait(credit_ccw, 1)
            r_cw = pltpu.make_async_remote_copy(
                src_ref=cw_ref.at[ss], dst_ref=cw_ref.at[rs],
                send_sem=send_cw.at[ss], recv_sem=recv_cw.at[rs],
                device_id=(right,), device_id_type=pl.DeviceIdType.MESH,
            )
            r_ccw = pltpu.make_async_remote_copy(
                src_ref=ccw_ref.at[ss], dst_ref=ccw_ref.at[rs],
                send_sem=send_ccw.at[ss], recv_sem=recv_ccw.at[rs],
                device_id=(left,), device_id_type=pl.DeviceIdType.MESH,
            )
            r_cw.start()
            r_ccw.start()

            r_cw.wait()
            if 1 <= s <= 5:
                pl.semaphore_signal(credit_cw, inc=1, device_id=(left,),
                                    device_id_type=pl.DeviceIdType.MESH)
            if s < N_DEV - 2:
                cw_ref[rs] = cw_ref[rs] + chunk(jnp.mod(d - 2 - s, N_DEV), 0)
            else:
                cw_ref[ss] = jnp.maximum(cw_ref[rs] + chunk(d, 0), 0.0)
                cp0 = pltpu.make_async_copy(
                    cw_ref.at[ss], out_ref.at[:, pl.ds(0, n_half)],
                    out_sems.at[0],
                )
                cp0.start()

            r_ccw.wait()
            if 1 <= s <= 5:
                pl.semaphore_signal(credit_ccw, inc=1, device_id=(right,),
                                    device_id_type=pl.DeviceIdType.MESH)
            if s < N_DEV - 2:
                ccw_ref[rs] = ccw_ref[rs] + chunk(jnp.mod(d + 2 + s, N_DEV), 1)
            else:
                ccw_ref[ss] = jnp.maximum(ccw_ref[rs] + chunk(d, 1), 0.0)
                cp1 = pltpu.make_async_copy(
                    ccw_ref.at[ss], out_ref.at[:, pl.ds(n_half, n_half)],
                    out_sems.at[1],
                )
                cp1.start()
                cp0.wait()
                cp1.wait()

    return pl.pallas_call(
        body,
        out_shape=jax.ShapeDtypeStruct((m_per, n_tot), jnp.float32),
        in_specs=[
            pl.BlockSpec(memory_space=pltpu.VMEM),
            pl.BlockSpec(memory_space=pltpu.VMEM),
        ],
        out_specs=pl.BlockSpec(memory_space=pl.ANY),
        scratch_shapes=[
            pltpu.VMEM((2, m_per, n_half), jnp.float32),
            pltpu.VMEM((2, m_per, n_half), jnp.float32),
            pltpu.SemaphoreType.DMA((2,)),
            pltpu.SemaphoreType.DMA((2,)),
            pltpu.SemaphoreType.DMA((2,)),
            pltpu.SemaphoreType.DMA((2,)),
            pltpu.SemaphoreType.REGULAR,
            pltpu.SemaphoreType.REGULAR,
            pltpu.SemaphoreType.DMA((2,)),
        ],
        compiler_params=pltpu.CompilerParams(
            collective_id=0,
            vmem_limit_bytes=64 * 1024 * 1024,
        ),
    )(x, w_mat)


# device time: 660586 ns/iter; 2.0516x vs baseline; 1.0818x over previous
import jax
import jax.numpy as jnp
from jax import lax
from jax.experimental import pallas as pl
from jax.experimental.pallas import tpu as pltpu

N_DEV = 8
NSUB = 4


def kernel(x, w_mat):
    m_tot, k_per = x.shape
    _, n_tot = w_mat.shape
    m_per = m_tot // N_DEV
    n_half = n_tot // 2
    n_sub = n_half // NSUB

    def body(x_ref, w_ref, out_ref, cw_ref, ccw_ref,
             send_cw, recv_cw, send_ccw, recv_ccw,
             credit_cw, credit_ccw, out_sems):
        d = lax.axis_index("i")
        right = jnp.mod(d + 1, N_DEV)
        left = jnp.mod(d - 1, N_DEV)

        barrier = pltpu.get_barrier_semaphore()
        for nbr in (left, right):
            pl.semaphore_signal(barrier, inc=1, device_id=(nbr,),
                                device_id_type=pl.DeviceIdType.MESH)
        pl.semaphore_wait(barrier, 2)

        def sub_dot(j, half, k):
            a = x_ref[pl.ds(j * m_per, m_per), :]
            lo = half * n_half + k * n_sub
            b = w_ref[:, lo:lo + n_sub]
            return jnp.dot(a, b, preferred_element_type=jnp.float32)

        def rdma(buf, ssl, rsl, ssem, rsem, dst, k):
            return pltpu.make_async_remote_copy(
                src_ref=buf.at[ssl, :, pl.ds(k * n_sub, n_sub)],
                dst_ref=buf.at[rsl, :, pl.ds(k * n_sub, n_sub)],
                send_sem=ssem.at[ssl, k],
                recv_sem=rsem.at[rsl, k],
                device_id=(dst,),
                device_id_type=pl.DeviceIdType.MESH,
            )

        pend_cw, pend_ccw = [], []
        for k in range(NSUB):
            cols = slice(k * n_sub, (k + 1) * n_sub)
            cw_ref[0, :, cols] = sub_dot(jnp.mod(d - 1, N_DEV), 0, k)
            r = rdma(cw_ref, 0, 1, send_cw, recv_cw, right, k)
            r.start()
            pend_cw.append(r)
            ccw_ref[0, :, cols] = sub_dot(jnp.mod(d + 1, N_DEV), 1, k)
            r = rdma(ccw_ref, 0, 1, send_ccw, recv_ccw, left, k)
            r.start()
            pend_ccw.append(r)

        out_pend = []
        for s in range(N_DEV - 1):
            ss, rs = s % 2, (s + 1) % 2
            last = s == N_DEV - 2
            nxt_cw, nxt_ccw = [], []
            for k in range(NSUB):
                cols = slice(k * n_sub, (k + 1) * n_sub)

                pend_cw[k].wait()
                if not last:
                    pl.semaphore_signal(credit_cw, inc=1, device_id=(left,),
                                        device_id_type=pl.DeviceIdType.MESH)
                    cw_ref[rs, :, cols] = (
                        cw_ref[rs, :, cols] + sub_dot(jnp.mod(d - 2 - s, N_DEV), 0, k)
                    )
                    pl.semaphore_wait(credit_cw, 1)
                    r = rdma(cw_ref, rs, ss, send_cw, recv_cw, right, k)
                    r.start()
                    nxt_cw.append(r)
                else:
                    cw_ref[ss, :, cols] = jnp.maximum(
                        cw_ref[rs, :, cols] + sub_dot(d, 0, k), 0.0
                    )
                    cp = pltpu.make_async_copy(
                        cw_ref.at[ss, :, pl.ds(k * n_sub, n_sub)],
                        out_ref.at[:, pl.ds(k * n_sub, n_sub)],
                        out_sems.at[0, k],
                    )
                    cp.start()
                    out_pend.append(cp)

                pend_ccw[k].wait()
                if not last:
                    pl.semaphore_signal(credit_ccw, inc=1, device_id=(right,),
                                        device_id_type=pl.DeviceIdType.MESH)
                    ccw_ref[rs, :, cols] = (
                        ccw_ref[rs, :, cols] + sub_dot(jnp.mod(d + 2 + s, N_DEV), 1, k)
                    )
                    pl.semaphore_wait(credit_ccw, 1)
                    r = rdma(ccw_ref, rs, ss, send_ccw, recv_ccw, left, k)
                    r.start()
                    nxt_ccw.append(r)
                else:
                    ccw_ref[ss, :, cols] = jnp.maximum(
                        ccw_ref[rs, :, cols] + sub_dot(d, 1, k), 0.0
                    )
                    cp = pltpu.make_async_copy(
                        ccw_ref.at[ss, :, pl.ds(k * n_sub, n_sub)],
                        out_ref.at[:, pl.ds(n_half + k * n_sub, n_sub)],
                        out_sems.at[1, k],
                    )
                    cp.start()
                    out_pend.append(cp)

            pend_cw, pend_ccw = nxt_cw, nxt_ccw

        for cp in out_pend:
            cp.wait()

    return pl.pallas_call(
        body,
        out_shape=jax.ShapeDtypeStruct((m_per, n_tot), jnp.float32),
        in_specs=[
            pl.BlockSpec(memory_space=pltpu.VMEM),
            pl.BlockSpec(memory_space=pltpu.VMEM),
        ],
        out_specs=pl.BlockSpec(memory_space=pl.ANY),
        scratch_shapes=[
            pltpu.VMEM((2, m_per, n_half), jnp.float32),
            pltpu.VMEM((2, m_per, n_half), jnp.float32),
            pltpu.SemaphoreType.DMA((2, NSUB)),
            pltpu.SemaphoreType.DMA((2, NSUB)),
            pltpu.SemaphoreType.DMA((2, NSUB)),
            pltpu.SemaphoreType.DMA((2, NSUB)),
            pltpu.SemaphoreType.REGULAR,
            pltpu.SemaphoreType.REGULAR,
            pltpu.SemaphoreType.DMA((2, NSUB)),
        ],
        compiler_params=pltpu.CompilerParams(
            collective_id=0,
            vmem_limit_bytes=64 * 1024 * 1024,
        ),
    )(x, w_mat)


# device time: 660129 ns/iter; 2.0530x vs baseline; 1.0007x over previous
import jax
import jax.numpy as jnp
from jax import lax
from jax.experimental import pallas as pl
from jax.experimental.pallas import tpu as pltpu

N_DEV = 8
NSUB = 8


def kernel(x, w_mat):
    m_tot, k_per = x.shape
    _, n_tot = w_mat.shape
    m_per = m_tot // N_DEV
    n_half = n_tot // 2
    n_sub = n_half // NSUB

    def body(x_ref, w_ref, out_ref, cw_ref, ccw_ref,
             send_cw, recv_cw, send_ccw, recv_ccw,
             credit_cw, credit_ccw, out_sems):
        d = lax.axis_index("i")
        right = jnp.mod(d + 1, N_DEV)
        left = jnp.mod(d - 1, N_DEV)

        barrier = pltpu.get_barrier_semaphore()
        for nbr in (left, right):
            pl.semaphore_signal(barrier, inc=1, device_id=(nbr,),
                                device_id_type=pl.DeviceIdType.MESH)
        pl.semaphore_wait(barrier, 2)

        def sub_dot(j, half, k):
            a = x_ref[pl.ds(j * m_per, m_per), :]
            lo = half * n_half + k * n_sub
            b = w_ref[:, lo:lo + n_sub]
            return jnp.dot(a, b, preferred_element_type=jnp.float32)

        def rdma(buf, ssl, rsl, ssem, rsem, dst, k):
            return pltpu.make_async_remote_copy(
                src_ref=buf.at[ssl, :, pl.ds(k * n_sub, n_sub)],
                dst_ref=buf.at[rsl, :, pl.ds(k * n_sub, n_sub)],
                send_sem=ssem.at[ssl, k],
                recv_sem=rsem.at[rsl, k],
                device_id=(dst,),
                device_id_type=pl.DeviceIdType.MESH,
            )

        pend_cw, pend_ccw = [], []
        for k in range(NSUB):
            cols = slice(k * n_sub, (k + 1) * n_sub)
            cw_ref[0, :, cols] = sub_dot(jnp.mod(d - 1, N_DEV), 0, k)
            r = rdma(cw_ref, 0, 1, send_cw, recv_cw, right, k)
            r.start()
            pend_cw.append(r)
            ccw_ref[0, :, cols] = sub_dot(jnp.mod(d + 1, N_DEV), 1, k)
            r = rdma(ccw_ref, 0, 1, send_ccw, recv_ccw, left, k)
            r.start()
            pend_ccw.append(r)

        out_pend = []
        for s in range(N_DEV - 1):
            ss, rs = s % 2, (s + 1) % 2
            last = s == N_DEV - 2
            nxt_cw, nxt_ccw = [], []
            for k in range(NSUB):
                cols = slice(k * n_sub, (k + 1) * n_sub)

                pend_cw[k].wait()
                if not last:
                    pl.semaphore_signal(credit_cw, inc=1, device_id=(left,),
                                        device_id_type=pl.DeviceIdType.MESH)
                    cw_ref[rs, :, cols] = (
                        cw_ref[rs, :, cols] + sub_dot(jnp.mod(d - 2 - s, N_DEV), 0, k)
                    )
                    pl.semaphore_wait(credit_cw, 1)
                    r = rdma(cw_ref, rs, ss, send_cw, recv_cw, right, k)
                    r.start()
                    nxt_cw.append(r)
                else:
                    cw_ref[ss, :, cols] = jnp.maximum(
                        cw_ref[rs, :, cols] + sub_dot(d, 0, k), 0.0
                    )
                    cp = pltpu.make_async_copy(
                        cw_ref.at[ss, :, pl.ds(k * n_sub, n_sub)],
                        out_ref.at[:, pl.ds(k * n_sub, n_sub)],
                        out_sems.at[0, k],
                    )
                    cp.start()
                    out_pend.append(cp)

                pend_ccw[k].wait()
                if not last:
                    pl.semaphore_signal(credit_ccw, inc=1, device_id=(right,),
                                        device_id_type=pl.DeviceIdType.MESH)
                    ccw_ref[rs, :, cols] = (
                        ccw_ref[rs, :, cols] + sub_dot(jnp.mod(d + 2 + s, N_DEV), 1, k)
                    )
                    pl.semaphore_wait(credit_ccw, 1)
                    r = rdma(ccw_ref, rs, ss, send_ccw, recv_ccw, left, k)
                    r.start()
                    nxt_ccw.append(r)
                else:
                    ccw_ref[ss, :, cols] = jnp.maximum(
                        ccw_ref[rs, :, cols] + sub_dot(d, 1, k), 0.0
                    )
                    cp = pltpu.make_async_copy(
                        ccw_ref.at[ss, :, pl.ds(k * n_sub, n_sub)],
                        out_ref.at[:, pl.ds(n_half + k * n_sub, n_sub)],
                        out_sems.at[1, k],
                    )
                    cp.start()
                    out_pend.append(cp)

            pend_cw, pend_ccw = nxt_cw, nxt_ccw

        for cp in out_pend:
            cp.wait()

    return pl.pallas_call(
        body,
        out_shape=jax.ShapeDtypeStruct((m_per, n_tot), jnp.float32),
        in_specs=[
            pl.BlockSpec(memory_space=pltpu.VMEM),
            pl.BlockSpec(memory_space=pltpu.VMEM),
        ],
        out_specs=pl.BlockSpec(memory_space=pl.ANY),
        scratch_shapes=[
            pltpu.VMEM((2, m_per, n_half), jnp.float32),
            pltpu.VMEM((2, m_per, n_half), jnp.float32),
            pltpu.SemaphoreType.DMA((2, NSUB)),
            pltpu.SemaphoreType.DMA((2, NSUB)),
            pltpu.SemaphoreType.DMA((2, NSUB)),
            pltpu.SemaphoreType.DMA((2, NSUB)),
            pltpu.SemaphoreType.REGULAR,
            pltpu.SemaphoreType.REGULAR,
            pltpu.SemaphoreType.DMA((2, NSUB)),
        ],
        compiler_params=pltpu.CompilerParams(
            collective_id=0,
            vmem_limit_bytes=64 * 1024 * 1024,
        ),
    )(x, w_mat)
